# baseline (speedup 1.0000x reference)
import functools

import jax
import jax.numpy as jnp
import numpy as np
from jax import lax
from jax.experimental import pallas as pl
from jax.experimental.pallas import tpu as pltpu
from jax.experimental.pallas import tpu_sc as plsc

NN = 10000
NB = 2
NT = NN * NB
E = 160000
NSUB = 16
CH = 128
CPT = 80
EPAD = NSUB * CPT * CH
NCHUNK = EPAD // CH
RPT = 640
RPT_LAST = NN - RPT * (NSUB - 1)
SINK = NN
ACC_ROWS = NN + 16
HID = 128
NL = 12
OUTF = 64
NBUF = 4
SLAB = 20
F2 = HID // 2
SCH = 64

_mesh = plsc.VectorSubcoreMesh(core_axis_name="c", subcore_axis_name="s")

_perm = np.zeros(HID, dtype=np.int64)
for _k in range(4):
    for _i in range(16):
        _perm[32 * _k + 2 * _i] = 32 * _k + _i
        _perm[32 * _k + 2 * _i + 1] = 32 * _k + 16 + _i
_P_np = np.zeros((HID, HID), dtype=np.float32)
_P_np[_perm, np.arange(HID)] = 1.0


def _copy_tile_rows(s, src_ref, src_base, dst_ref, dst_base):
    @pl.when(s < NSUB - 1)
    def _():
        pltpu.sync_copy(src_ref.at[pl.ds(src_base + s * RPT, RPT)],
                        dst_ref.at[pl.ds(dst_base + s * RPT, RPT)])

    @pl.when(s == NSUB - 1)
    def _():
        off = (NSUB - 1) * RPT
        pltpu.sync_copy(src_ref.at[pl.ds(src_base + off, RPT_LAST)],
                        dst_ref.at[pl.ds(dst_base + off, RPT_LAST)])


def _copy_tile_rows_via(s, src_ref, src_base, dst_ref, dst_base, tmp_ref):
    @pl.when(s < NSUB - 1)
    def _():
        pltpu.sync_copy(src_ref.at[pl.ds(src_base + s * RPT, RPT)],
                        tmp_ref.at[pl.ds(0, RPT)])
        pltpu.sync_copy(tmp_ref.at[pl.ds(0, RPT)],
                        dst_ref.at[pl.ds(dst_base + s * RPT, RPT)])

    @pl.when(s == NSUB - 1)
    def _():
        off = (NSUB - 1) * RPT
        pltpu.sync_copy(src_ref.at[pl.ds(src_base + off, RPT_LAST)],
                        tmp_ref.at[pl.ds(0, RPT_LAST)])
        pltpu.sync_copy(tmp_ref.at[pl.ds(0, RPT_LAST)],
                        dst_ref.at[pl.ds(dst_base + off, RPT_LAST)])


@functools.partial(
    pl.kernel,
    out_type=jax.ShapeDtypeStruct((NT,), jnp.float32),
    mesh=_mesh,
    scratch_types=[
        pltpu.VMEM((CPT, CH), jnp.int32),
        pltpu.VMEM((CH,), jnp.float32),
        pltpu.VMEM((RPT,), jnp.float32),
        pltpu.VMEM_SHARED((ACC_ROWS,), jnp.float32),
    ],
)
def _sc_degree(dst_hbm, ones_src_hbm, ones_init_hbm, z16_hbm, out_hbm,
               didx, ones_v, tmp, acc):
    c = lax.axis_index("c")
    s = lax.axis_index("s")
    pltpu.sync_copy(dst_hbm.at[c, pl.ds(s * CPT, CPT)], didx)
    pltpu.sync_copy(ones_src_hbm, ones_v)
    _copy_tile_rows_via(s, ones_init_hbm, 0, acc, 0, tmp)
    @pl.when(s == 0)
    def _():
        pltpu.sync_copy(z16_hbm, tmp.at[pl.ds(0, 16)])
        pltpu.sync_copy(tmp.at[pl.ds(0, 16)], acc.at[pl.ds(SINK, 16)])
    plsc.subcore_barrier()

    def body(j, _):
        pltpu.sync_copy(ones_v, acc.at[didx.at[j]], add=True)
        return 0
    lax.fori_loop(0, CPT, body, 0)
    plsc.subcore_barrier()
    _copy_tile_rows_via(s, acc, 0, out_hbm, c * NN, tmp)


@functools.partial(
    pl.kernel,
    out_type=jax.ShapeDtypeStruct((NT,), jnp.float32),
    mesh=_mesh,
    scratch_types=[
        pltpu.VMEM((CPT, CH), jnp.int32),
        pltpu.VMEM((CPT, CH), jnp.int32),
        pltpu.VMEM((4, CH), jnp.float32),
        pltpu.VMEM((RPT,), jnp.float32),
        pltpu.VMEM_SHARED((ACC_ROWS,), jnp.float32),
        pltpu.SemaphoreType.DMA((4,)),
    ],
)
def _sc_scalar_agg(src_hbm, dst_hbm, table_hbm, z16_hbm, out_hbm,
                   sidx, didx, buf, tmp, acc, sem):
    c = lax.axis_index("c")
    s = lax.axis_index("s")
    pltpu.sync_copy(src_hbm.at[c, pl.ds(s * CPT, CPT)], sidx)
    pltpu.sync_copy(dst_hbm.at[c, pl.ds(s * CPT, CPT)], didx)
    _copy_tile_rows_via(s, table_hbm, c * NN, acc, 0, tmp)
    @pl.when(s == 0)
    def _():
        pltpu.sync_copy(z16_hbm, tmp.at[pl.ds(0, 16)])
        pltpu.sync_copy(tmp.at[pl.ds(0, 16)], acc.at[pl.ds(SINK, 16)])
    plsc.subcore_barrier()

    for b in range(4):
        pltpu.async_copy(table_hbm.at[sidx.at[b]], buf.at[b], sem.at[b])

    def body(g, _):
        j0 = g * 4
        for b in range(4):
            pltpu.make_async_copy(
                table_hbm.at[sidx.at[j0 + b]], buf.at[b], sem.at[b]).wait()
            pltpu.sync_copy(buf.at[b], acc.at[didx.at[j0 + b]], add=True)
            @pl.when(j0 + b + 4 < CPT)
            def _():
                pltpu.async_copy(table_hbm.at[sidx.at[j0 + b + 4]],
                                 buf.at[b], sem.at[b])
        return 0
    lax.fori_loop(0, CPT // 4, body, 0)
    plsc.subcore_barrier()
    _copy_tile_rows_via(s, acc, 0, out_hbm, c * NN, tmp)


_M_HI = -65536


def _unpack_half(bufpk_b, q, buf32):
    def row(r, _):
        for k in range(4):
            w = bufpk_b[q * SCH + r, pl.ds(16 * k, 16)]
            lo = plsc.bitcast(jnp.left_shift(w, 16), jnp.float32)
            hi = plsc.bitcast(jnp.bitwise_and(w, _M_HI), jnp.float32)
            buf32[r, pl.ds(32 * k, 16)] = lo
            buf32[r, pl.ds(32 * k + 16, 16)] = hi
        return 0
    lax.fori_loop(0, SCH, row, 0)


@functools.partial(
    pl.kernel,
    out_type=jax.ShapeDtypeStruct((NT, HID), jnp.float32),
    mesh=_mesh,
    compiler_params=pltpu.CompilerParams(use_tc_tiling_on_sc=False,
                                         needs_layout_passes=False),
    scratch_types=[
        pltpu.VMEM((SLAB, CH), jnp.int32),
        pltpu.VMEM((2 * SLAB, SCH), jnp.int32),
        pltpu.VMEM((NBUF, CH, F2), jnp.int32),
        pltpu.VMEM((SCH, HID), jnp.float32),
        pltpu.VMEM_SHARED((ACC_ROWS, HID), jnp.float32),
        pltpu.SemaphoreType.DMA((NBUF,)),
    ],
)
def _sc_row_agg(src_hbm, dst_hbm, hpk_hbm, hf_hbm, z_hbm, out_hbm,
                sidx, didx, bufpk, buf32, acc, sem):
    c = lax.axis_index("c")
    s = lax.axis_index("s")
    _copy_tile_rows(s, hf_hbm, c * NN, acc, 0)
    @pl.when(s == 0)
    def _():
        pltpu.sync_copy(z_hbm, acc.at[pl.ds(SINK, 16)])
    plsc.subcore_barrier()

    for ih in range(CPT // SLAB):
        base = s * CPT + ih * SLAB
        pltpu.sync_copy(src_hbm.at[c, pl.ds(base, SLAB)], sidx)
        pltpu.sync_copy(dst_hbm.at[c, pl.ds(2 * base, 2 * SLAB)], didx)
        for b in range(NBUF):
            pltpu.async_copy(hpk_hbm.at[sidx.at[b]], bufpk.at[b], sem.at[b])

        def body(g, _):
            j0 = g * NBUF
            for b in range(NBUF):
                pltpu.make_async_copy(
                    hpk_hbm.at[sidx.at[j0 + b]], bufpk.at[b],
                    sem.at[b]).wait()
                _unpack_half(bufpk.at[b], 0, buf32)
                pltpu.sync_copy(buf32, acc.at[didx.at[2 * (j0 + b)]],
                                add=True)
                _unpack_half(bufpk.at[b], 1, buf32)
                @pl.when(j0 + b + NBUF < SLAB)
                def _():
                    pltpu.async_copy(hpk_hbm.at[sidx.at[j0 + b + NBUF]],
                                     bufpk.at[b], sem.at[b])
                pltpu.sync_copy(buf32, acc.at[didx.at[2 * (j0 + b) + 1]],
                                add=True)
            return 0
        lax.fori_loop(0, SLAB // NBUF, body, 0)
    plsc.subcore_barrier()
    _copy_tile_rows(s, acc, 0, out_hbm, c * NN)


def _leaky(v):
    return jnp.where(v >= 0.0, v, 0.01 * v)


def _tc_prep_body(deg_ref, x0_ref, dinv_ref, a_ref):
    dinv = lax.rsqrt(deg_ref[...])
    dinv_ref[...] = dinv
    a_ref[...] = x0_ref[...] * dinv


def _tc_layer1_body(t_ref, x0_ref, dinv_ref, w1_ref, b1_ref, wc_ref, p_ref,
                    x1_ref, h_ref, h16_ref):
    conv = (dinv_ref[...] * t_ref[...]) * w1_ref[...] + b1_ref[...]
    x1 = x0_ref[...] + _leaky(conv)
    x1_ref[...] = x1
    h = dinv_ref[...] * jnp.dot(
        x1, wc_ref[...], preferred_element_type=jnp.float32)
    h_ref[...] = h
    h16_ref[...] = jnp.dot(
        h, p_ref[...], preferred_element_type=jnp.float32
    ).astype(jnp.bfloat16)


def _tc_epilogue_body(acc_ref, xp_ref, dinv_ref, bc_ref, wc_ref, p_ref,
                      xn_ref, h_ref, h16_ref):
    conv = dinv_ref[...] * acc_ref[...] + bc_ref[...]
    xn = xp_ref[...] + _leaky(conv)
    xn_ref[...] = xn
    h = dinv_ref[...] * jnp.dot(
        xn, wc_ref[...], preferred_element_type=jnp.float32)
    h_ref[...] = h
    h16_ref[...] = jnp.dot(
        h, p_ref[...], preferred_element_type=jnp.float32
    ).astype(jnp.bfloat16)


def _tc_final_body(acc_ref, xp_ref, dinv_ref, bc_ref,
                   wg_ref, bg_ref, wb_ref, bb_ref, o_ref, pool_ref):
    i = pl.program_id(0)
    nprog = pl.num_programs(0)
    conv = dinv_ref[...] * acc_ref[...] + bc_ref[...]
    xn = xp_ref[...] + _leaky(conv)
    part = jnp.sum(xn, axis=0, keepdims=True) * (1.0 / NN)

    @pl.when(i == 0)
    def _():
        pool_ref[...] = jnp.zeros_like(pool_ref)

    half = nprog // NB

    @pl.when(i < half)
    def _():
        pool_ref[0:1, :] += part

    @pl.when(i >= half)
    def _():
        pool_ref[1:2, :] += part

    @pl.when(i == nprog - 1)
    def _():
        pooled = pool_ref[0:NB, :]
        gm = jnp.dot(pooled, wg_ref[...],
                     preferred_element_type=jnp.float32) + bg_ref[...]
        bt = jnp.dot(pooled, wb_ref[...],
                     preferred_element_type=jnp.float32) + bb_ref[...]
        o_ref[0, :, :] = gm
        o_ref[1, :, :] = bt


def _rows_spec(rows, cols):
    return pl.BlockSpec((rows, cols), lambda i: (i, 0))


def _full_spec(shape):
    nd = len(shape)
    return pl.BlockSpec(shape, lambda i: (0,) * nd)


def kernel(sst, nan_idx, edge_index_batch, batch_vec, W1, b1, Wc, bc, Wg, bg,
           Wb, bb):
    f32 = jnp.float32
    offs = (jnp.arange(NB, dtype=jnp.int32) * NN)[:, None]
    src = edge_index_batch[0].reshape(NB, E)
    dst = edge_index_batch[1].reshape(NB, E) - offs
    pad = EPAD - E
    src_p = jnp.concatenate(
        [src, jnp.zeros((NB, pad), jnp.int32)], axis=1).reshape(NB, NCHUNK, CH)
    dst_pad = jnp.concatenate(
        [dst, jnp.full((NB, pad), SINK, jnp.int32)], axis=1)
    dst_p = dst_pad.reshape(NB, NCHUNK, CH)
    dst_p2 = dst_pad.reshape(NB, 2 * NCHUNK, SCH)
    x0 = sst[:, :NN].reshape(NT, 1)
    ones_src = jnp.ones((CH,), f32)
    ones_init = jnp.ones((NN,), f32)
    z16 = jnp.zeros((16,), f32)
    zrow = jnp.zeros((16, HID), f32)
    pmat = jnp.asarray(_P_np)
    w1r = W1.reshape(1, HID)
    b1r = b1.reshape(1, HID)
    bcr = bc.reshape(3, 1, HID)
    wg2 = Wg.transpose(1, 0, 2).reshape(HID, NL * OUTF)
    bg2 = bg.reshape(1, NL * OUTF)
    wb2 = Wb.transpose(1, 0, 2).reshape(HID, NL * OUTF)
    bb2 = bb.reshape(1, NL * OUTF)

    deg = _sc_degree(dst_p, ones_src, ones_init, z16)

    grid1 = 10
    rows = NT // grid1
    dinv, a_col = pl.pallas_call(
        _tc_prep_body,
        grid=(grid1,),
        in_specs=[_rows_spec(rows, 1), _rows_spec(rows, 1)],
        out_specs=[_rows_spec(rows, 1), _rows_spec(rows, 1)],
        out_shape=[jax.ShapeDtypeStruct((NT, 1), f32),
                   jax.ShapeDtypeStruct((NT, 1), f32)],
    )(deg.reshape(NT, 1), x0)

    t = _sc_scalar_agg(src_p, dst_p, a_col.reshape(NT), z16)

    x1, h, h16 = pl.pallas_call(
        _tc_layer1_body,
        grid=(grid1,),
        in_specs=[_rows_spec(rows, 1), _rows_spec(rows, 1),
                  _rows_spec(rows, 1), _full_spec((1, HID)),
                  _full_spec((1, HID)), _full_spec((HID, HID)),
                  _full_spec((HID, HID))],
        out_specs=[_rows_spec(rows, HID), _rows_spec(rows, HID),
                   _rows_spec(rows, HID)],
        out_shape=[jax.ShapeDtypeStruct((NT, HID), f32),
                   jax.ShapeDtypeStruct((NT, HID), f32),
                   jax.ShapeDtypeStruct((NT, HID), jnp.bfloat16)],
    )(t.reshape(NT, 1), x0, dinv, w1r, b1r, Wc[0], pmat)

    xp = x1
    for i in range(2):
        hpk = lax.bitcast_convert_type(
            h16.reshape(NT, F2, 2), jnp.int32)
        acc = _sc_row_agg(src_p, dst_p2, hpk, h, zrow)
        xp, h, h16 = pl.pallas_call(
            _tc_epilogue_body,
            grid=(grid1,),
            in_specs=[_rows_spec(rows, HID), _rows_spec(rows, HID),
                      _rows_spec(rows, 1), _full_spec((1, HID)),
                      _full_spec((HID, HID)), _full_spec((HID, HID))],
            out_specs=[_rows_spec(rows, HID), _rows_spec(rows, HID),
                       _rows_spec(rows, HID)],
            out_shape=[jax.ShapeDtypeStruct((NT, HID), f32),
                       jax.ShapeDtypeStruct((NT, HID), f32),
                       jax.ShapeDtypeStruct((NT, HID), jnp.bfloat16)],
        )(acc, xp, dinv, bcr[i], Wc[i + 1], pmat)

    hpk = lax.bitcast_convert_type(h16.reshape(NT, F2, 2), jnp.int32)
    acc = _sc_row_agg(src_p, dst_p2, hpk, h, zrow)

    o = pl.pallas_call(
        _tc_final_body,
        grid=(grid1,),
        in_specs=[_rows_spec(rows, HID), _rows_spec(rows, HID),
                  _rows_spec(rows, 1), _full_spec((1, HID)),
                  _full_spec((HID, NL * OUTF)), _full_spec((1, NL * OUTF)),
                  _full_spec((HID, NL * OUTF)), _full_spec((1, NL * OUTF))],
        out_specs=pl.BlockSpec((2, NB, NL * OUTF), lambda i: (0, 0, 0)),
        out_shape=jax.ShapeDtypeStruct((2, NB, NL * OUTF), f32),
        scratch_shapes=[pltpu.VMEM((8, HID), f32)],
    )(acc, xp, dinv, bcr[2], wg2, bg2, wb2, bb2)

    gm = o[0].reshape(NB, NL, OUTF).transpose(1, 0, 2)
    bt = o[1].reshape(NB, NL, OUTF).transpose(1, 0, 2)
    return jnp.stack([gm, bt])


# speedup vs baseline: 1.1264x; 1.1264x over previous
import functools

import jax
import jax.numpy as jnp
from jax import lax
from jax.experimental import pallas as pl
from jax.experimental.pallas import tpu as pltpu
from jax.experimental.pallas import tpu_sc as plsc

NN = 10000
NB = 2
NT = NN * NB
E = 160000
NSUB = 16
CH = 128
CPT = 80
EPAD = NSUB * CPT * CH
NCHUNK = EPAD // CH
RPT = 640
RPT_LAST = NN - RPT * (NSUB - 1)
SINK = NN
ACC_ROWS = NN + 16
HID = 128
NL = 12
OUTF = 64
NBUF = 2
HALF = CPT // 2

_mesh = plsc.VectorSubcoreMesh(core_axis_name="c", subcore_axis_name="s")


def _copy_tile_rows(s, src_ref, src_base, dst_ref, dst_base):
    @pl.when(s < NSUB - 1)
    def _():
        pltpu.sync_copy(src_ref.at[pl.ds(src_base + s * RPT, RPT)],
                        dst_ref.at[pl.ds(dst_base + s * RPT, RPT)])

    @pl.when(s == NSUB - 1)
    def _():
        off = (NSUB - 1) * RPT
        pltpu.sync_copy(src_ref.at[pl.ds(src_base + off, RPT_LAST)],
                        dst_ref.at[pl.ds(dst_base + off, RPT_LAST)])


def _copy_tile_rows_via(s, src_ref, src_base, dst_ref, dst_base, tmp_ref):
    @pl.when(s < NSUB - 1)
    def _():
        pltpu.sync_copy(src_ref.at[pl.ds(src_base + s * RPT, RPT)],
                        tmp_ref.at[pl.ds(0, RPT)])
        pltpu.sync_copy(tmp_ref.at[pl.ds(0, RPT)],
                        dst_ref.at[pl.ds(dst_base + s * RPT, RPT)])

    @pl.when(s == NSUB - 1)
    def _():
        off = (NSUB - 1) * RPT
        pltpu.sync_copy(src_ref.at[pl.ds(src_base + off, RPT_LAST)],
                        tmp_ref.at[pl.ds(0, RPT_LAST)])
        pltpu.sync_copy(tmp_ref.at[pl.ds(0, RPT_LAST)],
                        dst_ref.at[pl.ds(dst_base + off, RPT_LAST)])


@functools.partial(
    pl.kernel,
    out_type=jax.ShapeDtypeStruct((NT,), jnp.float32),
    mesh=_mesh,
    scratch_types=[
        pltpu.VMEM((CPT, CH), jnp.int32),
        pltpu.VMEM((CH,), jnp.float32),
        pltpu.VMEM((RPT,), jnp.float32),
        pltpu.VMEM_SHARED((ACC_ROWS,), jnp.float32),
    ],
)
def _sc_degree(dst_hbm, ones_src_hbm, ones_init_hbm, z16_hbm, out_hbm,
               didx, ones_v, tmp, acc):
    c = lax.axis_index("c")
    s = lax.axis_index("s")
    pltpu.sync_copy(dst_hbm.at[c, pl.ds(s * CPT, CPT)], didx)
    pltpu.sync_copy(ones_src_hbm, ones_v)
    _copy_tile_rows_via(s, ones_init_hbm, 0, acc, 0, tmp)
    @pl.when(s == 0)
    def _():
        pltpu.sync_copy(z16_hbm, tmp.at[pl.ds(0, 16)])
        pltpu.sync_copy(tmp.at[pl.ds(0, 16)], acc.at[pl.ds(SINK, 16)])
    plsc.subcore_barrier()

    def body(j, _):
        pltpu.sync_copy(ones_v, acc.at[didx.at[j]], add=True)
        return 0
    lax.fori_loop(0, CPT, body, 0)
    plsc.subcore_barrier()
    _copy_tile_rows_via(s, acc, 0, out_hbm, c * NN, tmp)


@functools.partial(
    pl.kernel,
    out_type=jax.ShapeDtypeStruct((NT,), jnp.float32),
    mesh=_mesh,
    scratch_types=[
        pltpu.VMEM((CPT, CH), jnp.int32),
        pltpu.VMEM((CPT, CH), jnp.int32),
        pltpu.VMEM((8, CH), jnp.float32),
        pltpu.VMEM((RPT,), jnp.float32),
        pltpu.VMEM_SHARED((ACC_ROWS,), jnp.float32),
        pltpu.SemaphoreType.DMA((8,)),
    ],
)
def _sc_scalar_agg(src_hbm, dst_hbm, table_hbm, z16_hbm, out_hbm,
                   sidx, didx, buf, tmp, acc, sem):
    c = lax.axis_index("c")
    s = lax.axis_index("s")
    pltpu.sync_copy(src_hbm.at[c, pl.ds(s * CPT, CPT)], sidx)
    pltpu.sync_copy(dst_hbm.at[c, pl.ds(s * CPT, CPT)], didx)
    _copy_tile_rows_via(s, table_hbm, c * NN, acc, 0, tmp)
    @pl.when(s == 0)
    def _():
        pltpu.sync_copy(z16_hbm, tmp.at[pl.ds(0, 16)])
        pltpu.sync_copy(tmp.at[pl.ds(0, 16)], acc.at[pl.ds(SINK, 16)])
    plsc.subcore_barrier()

    for b in range(8):
        pltpu.async_copy(table_hbm.at[sidx.at[b]], buf.at[b], sem.at[b])

    def body(g, _):
        j0 = g * 8
        for b in range(8):
            pltpu.make_async_copy(
                table_hbm.at[sidx.at[j0 + b]], buf.at[b], sem.at[b]).wait()
            pltpu.sync_copy(buf.at[b], acc.at[didx.at[j0 + b]], add=True)
            @pl.when(j0 + b + 8 < CPT)
            def _():
                pltpu.async_copy(table_hbm.at[sidx.at[j0 + b + 8]],
                                 buf.at[b], sem.at[b])
        return 0
    lax.fori_loop(0, CPT // 8, body, 0)
    plsc.subcore_barrier()
    _copy_tile_rows_via(s, acc, 0, out_hbm, c * NN, tmp)


@functools.partial(
    pl.kernel,
    out_type=jax.ShapeDtypeStruct((NT, HID), jnp.float32),
    mesh=_mesh,
    scratch_types=[
        pltpu.VMEM((HALF, CH), jnp.int32),
        pltpu.VMEM((HALF, CH), jnp.int32),
        pltpu.VMEM((NBUF, CH, HID), jnp.float32),
        pltpu.VMEM_SHARED((ACC_ROWS, HID), jnp.float32),
        pltpu.SemaphoreType.DMA((NBUF,)),
    ],
)
def _sc_row_agg(src_hbm, dst_hbm, table_hbm, zrow_hbm, out_hbm,
                sidx, didx, buf, acc, sem):
    c = lax.axis_index("c")
    s = lax.axis_index("s")
    _copy_tile_rows(s, table_hbm, c * NN, acc, 0)
    @pl.when(s == 0)
    def _():
        pltpu.sync_copy(zrow_hbm, acc.at[pl.ds(SINK, 16)])
    plsc.subcore_barrier()

    for half in range(2):
        base = s * CPT + half * HALF
        pltpu.sync_copy(src_hbm.at[c, pl.ds(base, HALF)], sidx)
        pltpu.sync_copy(dst_hbm.at[c, pl.ds(base, HALF)], didx)
        for b in range(NBUF):
            pltpu.async_copy(table_hbm.at[sidx.at[b]], buf.at[b], sem.at[b])

        def body(g, _):
            j0 = g * NBUF
            for b in range(NBUF):
                pltpu.make_async_copy(
                    table_hbm.at[sidx.at[j0 + b]], buf.at[b], sem.at[b]).wait()
                pltpu.sync_copy(buf.at[b], acc.at[didx.at[j0 + b]], add=True)
                @pl.when(j0 + b + NBUF < HALF)
                def _():
                    pltpu.async_copy(table_hbm.at[sidx.at[j0 + b + NBUF]],
                                     buf.at[b], sem.at[b])
            return 0
        lax.fori_loop(0, HALF // NBUF, body, 0)
    plsc.subcore_barrier()
    _copy_tile_rows(s, acc, 0, out_hbm, c * NN)


def _leaky(v):
    return jnp.where(v >= 0.0, v, 0.01 * v)


def _tc_prep_body(deg_ref, x0_ref, dinv_ref, a_ref):
    dinv = lax.rsqrt(deg_ref[...])
    dinv_ref[...] = dinv
    a_ref[...] = x0_ref[...] * dinv


def _tc_layer1_body(t_ref, x0_ref, dinv_ref, w1_ref, b1_ref, wc_ref,
                    x1_ref, h1_ref):
    conv = (dinv_ref[...] * t_ref[...]) * w1_ref[...] + b1_ref[...]
    x1 = x0_ref[...] + _leaky(conv)
    x1_ref[...] = x1
    h1_ref[...] = dinv_ref[...] * jnp.dot(
        x1, wc_ref[...], preferred_element_type=jnp.float32)


def _tc_epilogue_body(acc_ref, xp_ref, dinv_ref, bc_ref, wc_ref,
                      xn_ref, hn_ref):
    conv = dinv_ref[...] * acc_ref[...] + bc_ref[...]
    xn = xp_ref[...] + _leaky(conv)
    xn_ref[...] = xn
    hn_ref[...] = dinv_ref[...] * jnp.dot(
        xn, wc_ref[...], preferred_element_type=jnp.float32)


def _tc_final_body(acc_ref, xp_ref, dinv_ref, bc_ref,
                   wg_ref, bg_ref, wb_ref, bb_ref, o_ref, pool_ref):
    i = pl.program_id(0)
    nprog = pl.num_programs(0)
    conv = dinv_ref[...] * acc_ref[...] + bc_ref[...]
    xn = xp_ref[...] + _leaky(conv)
    part = jnp.sum(xn, axis=0, keepdims=True) * (1.0 / NN)

    @pl.when(i == 0)
    def _():
        pool_ref[...] = jnp.zeros_like(pool_ref)

    half = nprog // NB

    @pl.when(i < half)
    def _():
        pool_ref[0:1, :] += part

    @pl.when(i >= half)
    def _():
        pool_ref[1:2, :] += part

    @pl.when(i == nprog - 1)
    def _():
        pooled = pool_ref[0:NB, :]
        gm = jnp.dot(pooled, wg_ref[...],
                     preferred_element_type=jnp.float32) + bg_ref[...]
        bt = jnp.dot(pooled, wb_ref[...],
                     preferred_element_type=jnp.float32) + bb_ref[...]
        o_ref[0, :, :] = gm
        o_ref[1, :, :] = bt


def _rows_spec(rows, cols):
    return pl.BlockSpec((rows, cols), lambda i: (i, 0))


def _full_spec(shape):
    nd = len(shape)
    return pl.BlockSpec(shape, lambda i: (0,) * nd)


def kernel(sst, nan_idx, edge_index_batch, batch_vec, W1, b1, Wc, bc, Wg, bg,
           Wb, bb):
    f32 = jnp.float32
    offs = (jnp.arange(NB, dtype=jnp.int32) * NN)[:, None]
    src = edge_index_batch[0].reshape(NB, E)
    dst = edge_index_batch[1].reshape(NB, E) - offs
    pad = EPAD - E
    src_p = jnp.concatenate(
        [src, jnp.zeros((NB, pad), jnp.int32)], axis=1).reshape(NB, NCHUNK, CH)
    dst_p = jnp.concatenate(
        [dst, jnp.full((NB, pad), SINK, jnp.int32)], axis=1).reshape(
            NB, NCHUNK, CH)
    x0 = sst[:, :NN].reshape(NT, 1)
    ones_src = jnp.ones((CH,), f32)
    ones_init = jnp.ones((NN,), f32)
    z16 = jnp.zeros((16,), f32)
    zrow = jnp.zeros((16, HID), f32)
    w1r = W1.reshape(1, HID)
    b1r = b1.reshape(1, HID)
    bcr = bc.reshape(3, 1, HID)
    wg2 = Wg.transpose(1, 0, 2).reshape(HID, NL * OUTF)
    bg2 = bg.reshape(1, NL * OUTF)
    wb2 = Wb.transpose(1, 0, 2).reshape(HID, NL * OUTF)
    bb2 = bb.reshape(1, NL * OUTF)

    deg = _sc_degree(dst_p, ones_src, ones_init, z16)

    grid1 = 10
    rows = NT // grid1
    dinv, a_col = pl.pallas_call(
        _tc_prep_body,
        grid=(grid1,),
        in_specs=[_rows_spec(rows, 1), _rows_spec(rows, 1)],
        out_specs=[_rows_spec(rows, 1), _rows_spec(rows, 1)],
        out_shape=[jax.ShapeDtypeStruct((NT, 1), f32),
                   jax.ShapeDtypeStruct((NT, 1), f32)],
    )(deg.reshape(NT, 1), x0)

    t = _sc_scalar_agg(src_p, dst_p, a_col.reshape(NT), z16)

    x1, h = pl.pallas_call(
        _tc_layer1_body,
        grid=(grid1,),
        in_specs=[_rows_spec(rows, 1), _rows_spec(rows, 1),
                  _rows_spec(rows, 1), _full_spec((1, HID)),
                  _full_spec((1, HID)), _full_spec((HID, HID))],
        out_specs=[_rows_spec(rows, HID), _rows_spec(rows, HID)],
        out_shape=[jax.ShapeDtypeStruct((NT, HID), f32),
                   jax.ShapeDtypeStruct((NT, HID), f32)],
    )(t.reshape(NT, 1), x0, dinv, w1r, b1r, Wc[0])

    xp = x1
    for i in range(2):
        acc = _sc_row_agg(src_p, dst_p, h, zrow)
        xp, h = pl.pallas_call(
            _tc_epilogue_body,
            grid=(grid1,),
            in_specs=[_rows_spec(rows, HID), _rows_spec(rows, HID),
                      _rows_spec(rows, 1), _full_spec((1, HID)),
                      _full_spec((HID, HID))],
            out_specs=[_rows_spec(rows, HID), _rows_spec(rows, HID)],
            out_shape=[jax.ShapeDtypeStruct((NT, HID), f32),
                       jax.ShapeDtypeStruct((NT, HID), f32)],
        )(acc, xp, dinv, bcr[i], Wc[i + 1])

    acc = _sc_row_agg(src_p, dst_p, h, zrow)

    o = pl.pallas_call(
        _tc_final_body,
        grid=(grid1,),
        in_specs=[_rows_spec(rows, HID), _rows_spec(rows, HID),
                  _rows_spec(rows, 1), _full_spec((1, HID)),
                  _full_spec((HID, NL * OUTF)), _full_spec((1, NL * OUTF)),
                  _full_spec((HID, NL * OUTF)), _full_spec((1, NL * OUTF))],
        out_specs=pl.BlockSpec((2, NB, NL * OUTF), lambda i: (0, 0, 0)),
        out_shape=jax.ShapeDtypeStruct((2, NB, NL * OUTF), f32),
        scratch_shapes=[pltpu.VMEM((8, HID), f32)],
    )(acc, xp, dinv, bcr[2], wg2, bg2, wb2, bb2)

    gm = o[0].reshape(NB, NL, OUTF).transpose(1, 0, 2)
    bt = o[1].reshape(NB, NL, OUTF).transpose(1, 0, 2)
    return jnp.stack([gm, bt])
